# baseline (speedup 1.0000x reference)
import functools

import jax
import jax.numpy as jnp
from jax import lax
from jax.experimental import pallas as pl
from jax.experimental.pallas import tpu as pltpu
from jax.experimental.pallas import tpu_sc as plsc

B = 4096
CTX = 20
EMB = 64
VOCAB = 100000

NW = 32
ROWS_PER_W = B // NW
ROWS_PER_CHUNK = 4
IDX_PER_CHUNK = ROWS_PER_CHUNK * CTX
NCHUNK = ROWS_PER_W // ROWS_PER_CHUNK
NSL = EMB // 16
SCALE = 1.0 / CTX


def _sc_pool_body(idx_hbm, table_hbm, out_hbm, idx_v, buf, acc, sem0, sem1):
    c = lax.axis_index("c")
    s = lax.axis_index("s")
    wid = s * 2 + c

    pltpu.sync_copy(idx_hbm.at[wid], idx_v)

    pltpu.async_copy(table_hbm.at[idx_v.at[0]], buf.at[0], sem0)
    pltpu.async_copy(table_hbm.at[idx_v.at[1]], buf.at[1], sem1)
    sems = (sem0, sem1)

    def outer(cc, carry):
        for nb in range(2):
            ch = cc * 2 + nb
            pltpu.make_async_copy(
                table_hbm.at[idx_v.at[ch]], buf.at[nb], sems[nb]
            ).wait()
            for r in range(ROWS_PER_CHUNK):
                for sl in range(NSL):
                    v = buf[nb, r * CTX, pl.ds(sl * 16, 16)]
                    for k in range(1, CTX):
                        v = v + buf[nb, r * CTX + k, pl.ds(sl * 16, 16)]
                    acc[ch * ROWS_PER_CHUNK + r, pl.ds(sl * 16, 16)] = v * SCALE

            @pl.when(ch + 2 < NCHUNK)
            def _():
                pltpu.async_copy(table_hbm.at[idx_v.at[ch + 2]], buf.at[nb], sems[nb])

        return carry

    lax.fori_loop(0, NCHUNK // 2, outer, 0)
    pltpu.sync_copy(acc, out_hbm.at[pl.ds(wid * ROWS_PER_W, ROWS_PER_W)])


@jax.jit
def _sc_pool(idx, table):
    mesh = plsc.VectorSubcoreMesh(core_axis_name="c", subcore_axis_name="s")
    return pl.kernel(
        _sc_pool_body,
        out_type=jax.ShapeDtypeStruct((B, EMB), jnp.float32),
        mesh=mesh,
        scratch_types=[
            pltpu.VMEM((NCHUNK, IDX_PER_CHUNK), jnp.int32),
            pltpu.VMEM((2, IDX_PER_CHUNK, EMB), jnp.float32),
            pltpu.VMEM((ROWS_PER_W, EMB), jnp.float32),
            pltpu.SemaphoreType.DMA,
            pltpu.SemaphoreType.DMA,
        ],
        compiler_params=pltpu.CompilerParams(use_tc_tiling_on_sc=False),
    )(idx, table)


BBLK = 1024
NBLK = 2048
NB = B // BBLK
NN = (VOCAB + NBLK - 1) // NBLK


def _mm_body(p_ref, w_ref, b_ref, o_ref):
    o_ref[...] = (
        jnp.dot(p_ref[...], w_ref[...], preferred_element_type=jnp.float32)
        + b_ref[...]
    )


@jax.jit
def _project(pooled, W, b2):
    return pl.pallas_call(
        _mm_body,
        grid=(NN, NB),
        in_specs=[
            pl.BlockSpec((BBLK, EMB), lambda j, i: (i, 0)),
            pl.BlockSpec((EMB, NBLK), lambda j, i: (0, j)),
            pl.BlockSpec((1, NBLK), lambda j, i: (0, j)),
        ],
        out_specs=pl.BlockSpec((BBLK, NBLK), lambda j, i: (i, j)),
        out_shape=jax.ShapeDtypeStruct((B, VOCAB), jnp.float32),
    )(pooled, W, b2)


def kernel(x, emb_table, W, b):
    idx = x.astype(jnp.int32).reshape(NW, NCHUNK, IDX_PER_CHUNK)
    pooled = _sc_pool(idx, emb_table)
    return _project(pooled, W, b.reshape(1, VOCAB))


# speedup vs baseline: 1.0109x; 1.0109x over previous
import functools

import jax
import jax.numpy as jnp
from jax import lax
from jax.experimental import pallas as pl
from jax.experimental.pallas import tpu as pltpu
from jax.experimental.pallas import tpu_sc as plsc

B = 4096
CTX = 20
EMB = 64
VOCAB = 100000

NW = 32
ROWS_PER_W = B // NW
ROWS_PER_CHUNK = 4
IDX_PER_CHUNK = ROWS_PER_CHUNK * CTX
NCHUNK = ROWS_PER_W // ROWS_PER_CHUNK
NSL = EMB // 16
SCALE = 1.0 / CTX


def _sc_pool_body(idx_hbm, table_hbm, out_hbm, idx_v, buf, acc, sem0, sem1):
    c = lax.axis_index("c")
    s = lax.axis_index("s")
    wid = s * 2 + c

    pltpu.sync_copy(idx_hbm.at[wid], idx_v)

    pltpu.async_copy(table_hbm.at[idx_v.at[0]], buf.at[0], sem0)
    pltpu.async_copy(table_hbm.at[idx_v.at[1]], buf.at[1], sem1)
    sems = (sem0, sem1)

    def outer(cc, carry):
        for nb in range(2):
            ch = cc * 2 + nb
            pltpu.make_async_copy(
                table_hbm.at[idx_v.at[ch]], buf.at[nb], sems[nb]
            ).wait()
            for r in range(ROWS_PER_CHUNK):
                for sl in range(NSL):
                    v = buf[nb, r * CTX, pl.ds(sl * 16, 16)]
                    for k in range(1, CTX):
                        v = v + buf[nb, r * CTX + k, pl.ds(sl * 16, 16)]
                    acc[ch * ROWS_PER_CHUNK + r, pl.ds(sl * 16, 16)] = v * SCALE

            @pl.when(ch + 2 < NCHUNK)
            def _():
                pltpu.async_copy(table_hbm.at[idx_v.at[ch + 2]], buf.at[nb], sems[nb])

        return carry

    lax.fori_loop(0, NCHUNK // 2, outer, 0)
    pltpu.sync_copy(acc, out_hbm.at[pl.ds(wid * ROWS_PER_W, ROWS_PER_W)])


@jax.jit
def _sc_pool(idx, table):
    mesh = plsc.VectorSubcoreMesh(core_axis_name="c", subcore_axis_name="s")
    return pl.kernel(
        _sc_pool_body,
        out_type=jax.ShapeDtypeStruct((B, EMB), jnp.float32),
        mesh=mesh,
        scratch_types=[
            pltpu.VMEM((NCHUNK, IDX_PER_CHUNK), jnp.int32),
            pltpu.VMEM((2, IDX_PER_CHUNK, EMB), jnp.float32),
            pltpu.VMEM((ROWS_PER_W, EMB), jnp.float32),
            pltpu.SemaphoreType.DMA,
            pltpu.SemaphoreType.DMA,
        ],
        compiler_params=pltpu.CompilerParams(use_tc_tiling_on_sc=False),
    )(idx, table)


BBLK = 1024
NBLK = 2048
NB = B // BBLK
NN = (VOCAB + NBLK - 1) // NBLK


def _mm_body(p_ref, w_ref, b_ref, o_ref):
    o_ref[...] = (
        jnp.dot(p_ref[...], w_ref[...], preferred_element_type=jnp.float32)
        + b_ref[...]
    )


@jax.jit
def _project(pooled, W, b2):
    return pl.pallas_call(
        _mm_body,
        grid=(NN, NB),
        in_specs=[
            pl.BlockSpec((BBLK, EMB), lambda j, i: (i, 0)),
            pl.BlockSpec((EMB, NBLK), lambda j, i: (0, j)),
            pl.BlockSpec((1, NBLK), lambda j, i: (0, j)),
        ],
        out_specs=pl.BlockSpec((BBLK, NBLK), lambda j, i: (i, j)),
        out_shape=jax.ShapeDtypeStruct((B, VOCAB), jnp.float32),
    )(pooled, W, b2)


def kernel(x, emb_table, W, b):
    idx = x.astype(jnp.int32).reshape(NW, NCHUNK, IDX_PER_CHUNK)
    pooled = _sc_pool(idx, emb_table)
    return _project(
        pooled.astype(jnp.bfloat16), W.astype(jnp.bfloat16), b.reshape(1, VOCAB)
    )


# speedup vs baseline: 1.0201x; 1.0091x over previous
import functools

import jax
import jax.numpy as jnp
from jax import lax
from jax.experimental import pallas as pl
from jax.experimental.pallas import tpu as pltpu
from jax.experimental.pallas import tpu_sc as plsc

B = 4096
CTX = 20
EMB = 64
VOCAB = 100000

NW = 32
ROWS_PER_W = B // NW
ROWS_PER_CHUNK = 4
IDX_PER_CHUNK = ROWS_PER_CHUNK * CTX
NCHUNK = ROWS_PER_W // ROWS_PER_CHUNK
NSL = EMB // 16
SCALE = 1.0 / CTX


def _sc_pool_body(idx_hbm, table_hbm, out_hbm, idx_v, buf, acc, sem0, sem1):
    c = lax.axis_index("c")
    s = lax.axis_index("s")
    wid = s * 2 + c

    pltpu.sync_copy(idx_hbm.at[wid], idx_v)

    pltpu.async_copy(table_hbm.at[idx_v.at[0]], buf.at[0], sem0)
    pltpu.async_copy(table_hbm.at[idx_v.at[1]], buf.at[1], sem1)
    sems = (sem0, sem1)

    def outer(cc, carry):
        for nb in range(2):
            ch = cc * 2 + nb
            pltpu.make_async_copy(
                table_hbm.at[idx_v.at[ch]], buf.at[nb], sems[nb]
            ).wait()
            for r in range(ROWS_PER_CHUNK):
                for sl in range(NSL):
                    v = buf[nb, r * CTX, pl.ds(sl * 16, 16)]
                    for k in range(1, CTX):
                        v = v + buf[nb, r * CTX + k, pl.ds(sl * 16, 16)]
                    acc[ch * ROWS_PER_CHUNK + r, pl.ds(sl * 16, 16)] = v * SCALE

            @pl.when(ch + 2 < NCHUNK)
            def _():
                pltpu.async_copy(table_hbm.at[idx_v.at[ch + 2]], buf.at[nb], sems[nb])

        return carry

    lax.fori_loop(0, NCHUNK // 2, outer, 0)
    pltpu.sync_copy(acc, out_hbm.at[pl.ds(wid * ROWS_PER_W, ROWS_PER_W)])


@jax.jit
def _sc_pool(idx, table):
    mesh = plsc.VectorSubcoreMesh(core_axis_name="c", subcore_axis_name="s")
    return pl.kernel(
        _sc_pool_body,
        out_type=jax.ShapeDtypeStruct((B, EMB), jnp.float32),
        mesh=mesh,
        scratch_types=[
            pltpu.VMEM((NCHUNK, IDX_PER_CHUNK), jnp.int32),
            pltpu.VMEM((2, IDX_PER_CHUNK, EMB), jnp.float32),
            pltpu.VMEM((ROWS_PER_W, EMB), jnp.float32),
            pltpu.SemaphoreType.DMA,
            pltpu.SemaphoreType.DMA,
        ],
        compiler_params=pltpu.CompilerParams(use_tc_tiling_on_sc=False),
    )(idx, table)


BBLK = 1024
NBLK = 4096
NB = B // BBLK
NN = (VOCAB + NBLK - 1) // NBLK


def _mm_body(p_ref, w_ref, b_ref, o_ref):
    o_ref[...] = (
        jnp.dot(p_ref[...], w_ref[...], preferred_element_type=jnp.float32)
        + b_ref[...]
    )


@jax.jit
def _project(pooled, W, b2):
    return pl.pallas_call(
        _mm_body,
        grid=(NN, NB),
        in_specs=[
            pl.BlockSpec((BBLK, EMB), lambda j, i: (i, 0)),
            pl.BlockSpec((EMB, NBLK), lambda j, i: (0, j)),
            pl.BlockSpec((1, NBLK), lambda j, i: (0, j)),
        ],
        out_specs=pl.BlockSpec((BBLK, NBLK), lambda j, i: (i, j)),
        out_shape=jax.ShapeDtypeStruct((B, VOCAB), jnp.float32),
    )(pooled, W, b2)


def kernel(x, emb_table, W, b):
    idx = x.astype(jnp.int32).reshape(NW, NCHUNK, IDX_PER_CHUNK)
    pooled = _sc_pool(idx, emb_table)
    return _project(
        pooled.astype(jnp.bfloat16), W.astype(jnp.bfloat16), b.reshape(1, VOCAB)
    )


# speedup vs baseline: 3.4262x; 3.3587x over previous
import jax
import jax.numpy as jnp
from jax import lax
from jax.experimental import pallas as pl
from jax.experimental.pallas import tpu as pltpu
from jax.experimental.pallas import tpu_sc as plsc

B = 4096
CTX = 20
EMB = 64
VOCAB = 100000

NW = 32
ROWS_PER_W = B // NW
ROWS_PER_CHUNK = 4
IDX_PER_CHUNK = ROWS_PER_CHUNK * CTX
NCHUNK = ROWS_PER_W // ROWS_PER_CHUNK
NSL = EMB // 16
SCALE = 1.0 / CTX


NBUF = 2


def _sc_pool_body(idx_hbm, table_hbm, out_hbm, idx_v, buf, acc, *sems):
    c = lax.axis_index("c")
    s = lax.axis_index("s")
    wid = s * 2 + c

    pltpu.sync_copy(idx_hbm.at[wid], idx_v)

    for nb in range(NBUF):
        pltpu.async_copy(table_hbm.at[idx_v.at[nb]], buf.at[nb], sems[nb])

    def outer(cc, carry):
        for nb in range(NBUF):
            ch = cc * NBUF + nb
            pltpu.make_async_copy(
                table_hbm.at[idx_v.at[ch]], buf.at[nb], sems[nb]
            ).wait()
            for r in range(ROWS_PER_CHUNK):
                for sl in range(NSL):
                    v = buf[nb, r * CTX, pl.ds(sl * 16, 16)]
                    for k in range(1, CTX):
                        v = v + buf[nb, r * CTX + k, pl.ds(sl * 16, 16)]
                    acc[ch * ROWS_PER_CHUNK + r, pl.ds(sl * 16, 16)] = v * SCALE

            @pl.when(ch + NBUF < NCHUNK)
            def _():
                pltpu.async_copy(
                    table_hbm.at[idx_v.at[ch + NBUF]], buf.at[nb], sems[nb]
                )

        return carry

    lax.fori_loop(0, NCHUNK // NBUF, outer, 0)
    pltpu.sync_copy(acc, out_hbm.at[pl.ds(wid * ROWS_PER_W, ROWS_PER_W)])


@jax.jit
def _sc_pool(idx, table):
    mesh = plsc.VectorSubcoreMesh(core_axis_name="c", subcore_axis_name="s")
    return pl.kernel(
        _sc_pool_body,
        out_type=jax.ShapeDtypeStruct((B, EMB), jnp.float32),
        mesh=mesh,
        scratch_types=[
            pltpu.VMEM((NCHUNK, IDX_PER_CHUNK), jnp.int32),
            pltpu.VMEM((NBUF, IDX_PER_CHUNK, EMB), jnp.float32),
            pltpu.VMEM((ROWS_PER_W, EMB), jnp.float32),
        ]
        + [pltpu.SemaphoreType.DMA] * NBUF,
        compiler_params=pltpu.CompilerParams(use_tc_tiling_on_sc=False),
    )(idx, table)


NBLK = 1024
NN = (VOCAB + NBLK - 1) // NBLK


def _mm_body(p_ref, w_ref, b_ref, o_ref):
    acc = jax.lax.dot_general(
        w_ref[...].astype(jnp.bfloat16),
        p_ref[...],
        (((0,), (1,)), ((), ())),
        preferred_element_type=jnp.float32,
    )
    o_ref[...] = acc + b_ref[...].T


@jax.jit
def _project_t(pooled, W, b2):
    return pl.pallas_call(
        _mm_body,
        grid=(NN,),
        in_specs=[
            pl.BlockSpec((B, EMB), lambda j: (0, 0)),
            pl.BlockSpec((EMB, NBLK), lambda j: (0, j)),
            pl.BlockSpec((1, NBLK), lambda j: (0, j)),
        ],
        out_specs=pl.BlockSpec((NBLK, B), lambda j: (j, 0)),
        out_shape=jax.ShapeDtypeStruct((VOCAB, B), jnp.float32),
    )(pooled, W, b2)


def kernel(x, emb_table, W, b):
    idx = x.astype(jnp.int32).reshape(NW, NCHUNK, IDX_PER_CHUNK)
    pooled = _sc_pool(idx, emb_table)
    out_t = _project_t(pooled.astype(jnp.bfloat16), W, b.reshape(1, VOCAB))
    return out_t.T
